# Initial kernel scaffold; baseline (speedup 1.0000x reference)
import jax
import jax.numpy as jnp
from jax.experimental import pallas as pl


def kernel(x, edge_index, W1, b1, W2, b2):
    raise NotImplementedError("write your pallas kernel here")


# speedup vs baseline: 273.2878x; 273.2878x over previous
import functools

import jax
import jax.numpy as jnp
from jax import lax
from jax.experimental import pallas as pl
from jax.experimental.pallas import tpu as pltpu
from jax.experimental.pallas import tpu_sc as plsc

NC = 2
NS = 16
NW = NC * NS


def _pad_up(v, m):
    return (v + m - 1) // m * m


@functools.lru_cache(maxsize=None)
def _deg_kernel(Np, Ep, CH):
    SLN = Np // NS
    nch = Ep // (NW * CH)
    ew = nch * CH
    mesh = plsc.VectorSubcoreMesh(
        core_axis_name="c", subcore_axis_name="s", num_cores=NC, num_subcores=NS)

    @functools.partial(
        pl.kernel,
        out_type=jax.ShapeDtypeStruct((NC * Np,), jnp.float32),
        mesh=mesh,
        scratch_types=[
            pltpu.VMEM_SHARED((Np,), jnp.float32),
            pltpu.VMEM((CH,), jnp.int32),
            pltpu.VMEM((CH,), jnp.float32),
            pltpu.VMEM((SLN,), jnp.float32),
        ],
    )
    def deg(dst_hbm, ones_hbm, zeros_hbm, out_hbm, acc, idx, ones_v, bounce):
        c = lax.axis_index("c")
        s = lax.axis_index("s")
        wid = c * NS + s
        row = pl.multiple_of(s * SLN, 8)
        rsl = pl.ds(row, SLN)
        pltpu.sync_copy(ones_hbm, ones_v)
        pltpu.sync_copy(zeros_hbm, bounce)
        pltpu.sync_copy(bounce, acc.at[rsl])
        plsc.subcore_barrier()

        base = wid * ew

        def chunk(k, carry):
            off = pl.multiple_of(base + k * CH, 8)
            pltpu.sync_copy(dst_hbm.at[pl.ds(off, CH)], idx)
            pltpu.sync_copy(ones_v, acc.at[idx], add=True)
            return carry

        lax.fori_loop(0, nch, chunk, 0)
        plsc.subcore_barrier()
        pltpu.sync_copy(acc.at[rsl], bounce)
        orow = pl.multiple_of(c * Np + row, 8)
        pltpu.sync_copy(bounce, out_hbm.at[pl.ds(orow, SLN)])

    return deg


@functools.lru_cache(maxsize=None)
def _edge_kernel(B, Np, Ep, CH):
    SLN = Np // NS
    nch = Ep // (NW * CH)
    ew = nch * CH
    mesh = plsc.VectorSubcoreMesh(
        core_axis_name="c", subcore_axis_name="s", num_cores=NC, num_subcores=NS)

    scratch = ([pltpu.VMEM_SHARED((Np,), jnp.float32)] * B
               + [pltpu.VMEM_SHARED((Np,), jnp.float32)] * B
               + [pltpu.VMEM((CH,), jnp.int32),
                  pltpu.VMEM((CH,), jnp.int32),
                  pltpu.VMEM((CH,), jnp.float32),
                  pltpu.VMEM((SLN,), jnp.float32)])

    @functools.partial(
        pl.kernel,
        out_type=jax.ShapeDtypeStruct((NC * B * Np,), jnp.float32),
        mesh=mesh,
        scratch_types=scratch,
    )
    def edge(y_hbm, src_hbm, dst_hbm, zeros_hbm, out_hbm, *refs):
        tabs = refs[:B]
        accs = refs[B:2 * B]
        isrc, idst, msg, bounce = refs[2 * B:]
        c = lax.axis_index("c")
        s = lax.axis_index("s")
        wid = c * NS + s
        row = pl.multiple_of(s * SLN, 8)
        rsl = pl.ds(row, SLN)

        pltpu.sync_copy(zeros_hbm, bounce)
        for b in range(B):
            pltpu.sync_copy(bounce, accs[b].at[rsl])
        for b in range(B):
            yrow = pl.multiple_of(b * Np + row, 8)
            pltpu.sync_copy(y_hbm.at[pl.ds(yrow, SLN)], bounce)
            pltpu.sync_copy(bounce, tabs[b].at[rsl])
        plsc.subcore_barrier()

        base = wid * ew

        def chunk(k, carry):
            off = pl.multiple_of(base + k * CH, 8)
            pltpu.sync_copy(src_hbm.at[pl.ds(off, CH)], isrc)
            pltpu.sync_copy(dst_hbm.at[pl.ds(off, CH)], idst)
            for b in range(B):
                pltpu.sync_copy(tabs[b].at[isrc], msg)
                pltpu.sync_copy(msg, accs[b].at[idst], add=True)
            return carry

        lax.fori_loop(0, nch, chunk, 0)
        plsc.subcore_barrier()
        for b in range(B):
            pltpu.sync_copy(accs[b].at[rsl], bounce)
            orow = pl.multiple_of((c * B + b) * Np + row, 8)
            pltpu.sync_copy(bounce, out_hbm.at[pl.ds(orow, SLN)])

    return edge


def _dense_pre(degp, xp):
    def body(degp_ref, xp_ref, y_ref, dinv_ref):
        d = degp_ref[0:1, :] + degp_ref[1:2, :] + 1.0
        dinv = lax.rsqrt(d)
        dinv_ref[...] = dinv
        y_ref[...] = xp_ref[...] * dinv

    return pl.pallas_call(
        body,
        out_shape=(jax.ShapeDtypeStruct(xp.shape, jnp.float32),
                   jax.ShapeDtypeStruct((1, xp.shape[1]), jnp.float32)),
    )(degp, xp)


def _dense_mid(aggp, dinv, xp, w1, b1, w2):
    H = w1.shape[0]

    def body(aggp_ref, dinv_ref, xp_ref, w1_ref, b1_ref, w2_ref, t_ref):
        dv = dinv_ref[...]
        a = (aggp_ref[0] + aggp_ref[1] + xp_ref[...] * dv) * dv

        def jb(j, acc):
            return acc + jnp.maximum(a * w1_ref[j] + b1_ref[j], 0.0) * w2_ref[j]

        sv = lax.fori_loop(0, H, jb, jnp.zeros_like(a))
        t_ref[...] = sv * dv

    smem = pl.BlockSpec(memory_space=pltpu.SMEM)
    vmem = pl.BlockSpec(memory_space=pltpu.VMEM)
    return pl.pallas_call(
        body,
        in_specs=[vmem, vmem, vmem, smem, smem, smem],
        out_shape=jax.ShapeDtypeStruct(xp.shape, jnp.float32),
    )(aggp, dinv, xp, w1, b1, w2)


def _dense_fini(outp, dinv, t, b2):
    def body(outp_ref, dinv_ref, t_ref, b2_ref, o_ref):
        dv = dinv_ref[...]
        o_ref[...] = (outp_ref[0] + outp_ref[1] + t_ref[...]) * dv + b2_ref[0]

    smem = pl.BlockSpec(memory_space=pltpu.SMEM)
    vmem = pl.BlockSpec(memory_space=pltpu.VMEM)
    return pl.pallas_call(
        body,
        in_specs=[vmem, vmem, vmem, smem],
        out_shape=jax.ShapeDtypeStruct(t.shape, jnp.float32),
    )(outp, dinv, t, b2)


def kernel(x, edge_index, W1, b1, W2, b2):
    B, N = x.shape
    E = edge_index.shape[1]
    Np = (N // 256 + 1) * 256
    CH = 1000

    Ep = _pad_up(E, NW * CH)
    if Ep != E:
        pad = jnp.full((2, Ep - E), N, dtype=edge_index.dtype)
        e = jnp.concatenate([edge_index, pad], axis=1)
    else:
        e = edge_index
    src = e[0].reshape(-1)
    dst = e[1].reshape(-1)

    xp = jnp.pad(x.astype(jnp.float32), ((0, 0), (0, Np - N)))
    zeros = jnp.zeros((Np // NS,), jnp.float32)
    ones = jnp.ones((CH,), jnp.float32)

    degp = _deg_kernel(Np, Ep, CH)(dst, ones, zeros).reshape(NC, Np)
    y, dinv = _dense_pre(degp, xp)
    aggp = _edge_kernel(B, Np, Ep, CH)(
        y.reshape(-1), src, dst, zeros).reshape(NC, B, Np)
    t = _dense_mid(aggp, dinv, xp, W1.reshape(-1), b1.reshape(-1),
                   W2.reshape(-1))
    outp = _edge_kernel(B, Np, Ep, CH)(
        t.reshape(-1), src, dst, zeros).reshape(NC, B, Np)
    o = _dense_fini(outp, dinv, t, b2.reshape(-1))
    return o[:, :N]


# speedup vs baseline: 439.7469x; 1.6091x over previous
import functools

import jax
import jax.numpy as jnp
from jax import lax
from jax.experimental import pallas as pl
from jax.experimental.pallas import tpu as pltpu
from jax.experimental.pallas import tpu_sc as plsc

NC = 2
NS = 16
NW = NC * NS


def _pad_up(v, m):
    return (v + m - 1) // m * m


@functools.lru_cache(maxsize=None)
def _deg_kernel(Np, Ep, CH):
    SLN = Np // NS
    nch = Ep // (NW * CH)
    ew = nch * CH
    mesh = plsc.VectorSubcoreMesh(
        core_axis_name="c", subcore_axis_name="s", num_cores=NC, num_subcores=NS)

    @functools.partial(
        pl.kernel,
        out_type=jax.ShapeDtypeStruct((NC * Np,), jnp.float32),
        mesh=mesh,
        scratch_types=[
            pltpu.VMEM_SHARED((Np,), jnp.float32),
            pltpu.VMEM((CH,), jnp.int32),
            pltpu.VMEM((CH,), jnp.float32),
            pltpu.VMEM((SLN,), jnp.float32),
        ],
    )
    def deg(dst_hbm, ones_hbm, zeros_hbm, out_hbm, acc, idx, ones_v, bounce):
        c = lax.axis_index("c")
        s = lax.axis_index("s")
        wid = c * NS + s
        row = pl.multiple_of(s * SLN, 8)
        rsl = pl.ds(row, SLN)
        pltpu.sync_copy(ones_hbm, ones_v)
        pltpu.sync_copy(zeros_hbm, bounce)
        pltpu.sync_copy(bounce, acc.at[rsl])
        plsc.subcore_barrier()

        base = wid * ew

        def chunk(k, carry):
            off = pl.multiple_of(base + k * CH, 8)
            pltpu.sync_copy(dst_hbm.at[pl.ds(off, CH)], idx)
            pltpu.sync_copy(ones_v, acc.at[idx], add=True)
            return carry

        lax.fori_loop(0, nch, chunk, 0)
        plsc.subcore_barrier()
        pltpu.sync_copy(acc.at[rsl], bounce)
        orow = pl.multiple_of(c * Np + row, 8)
        pltpu.sync_copy(bounce, out_hbm.at[pl.ds(orow, SLN)])

    return deg


@functools.lru_cache(maxsize=None)
def _edge_kernel(B, Np, Ep, CH):
    SLN = Np // NS
    nch = Ep // (NW * CH)
    ew = nch * CH
    mesh = plsc.VectorSubcoreMesh(
        core_axis_name="c", subcore_axis_name="s", num_cores=NC, num_subcores=NS)

    scratch = ([pltpu.VMEM_SHARED((Np,), jnp.float32)] * B
               + [pltpu.VMEM_SHARED((Np,), jnp.float32)] * B
               + [pltpu.VMEM((CH,), jnp.float32)] * B
               + [pltpu.VMEM((CH,), jnp.int32),
                  pltpu.VMEM((CH,), jnp.int32),
                  pltpu.VMEM((SLN,), jnp.float32),
                  pltpu.SemaphoreType.DMA,
                  pltpu.SemaphoreType.DMA,
                  pltpu.SemaphoreType.DMA])

    @functools.partial(
        pl.kernel,
        out_type=jax.ShapeDtypeStruct((NC * B * Np,), jnp.float32),
        mesh=mesh,
        scratch_types=scratch,
    )
    def edge(y_hbm, src_hbm, dst_hbm, zeros_hbm, out_hbm, *refs):
        tabs = refs[:B]
        accs = refs[B:2 * B]
        msgs = refs[2 * B:3 * B]
        isrc, idst, bounce, sem_i, sem_g, sem_s = refs[3 * B:]
        c = lax.axis_index("c")
        s = lax.axis_index("s")
        wid = c * NS + s
        row = pl.multiple_of(s * SLN, 8)
        rsl = pl.ds(row, SLN)

        pltpu.sync_copy(zeros_hbm, bounce)
        for b in range(B):
            pltpu.sync_copy(bounce, accs[b].at[rsl])
        for b in range(B):
            yrow = pl.multiple_of(b * Np + row, 8)
            pltpu.sync_copy(y_hbm.at[pl.ds(yrow, SLN)], bounce)
            pltpu.sync_copy(bounce, tabs[b].at[rsl])
        plsc.subcore_barrier()

        base = wid * ew

        def chunk(k, carry):
            off = pl.multiple_of(base + k * CH, 8)
            di = pltpu.async_copy(src_hbm.at[pl.ds(off, CH)], isrc, sem_i)
            di2 = pltpu.async_copy(dst_hbm.at[pl.ds(off, CH)], idst, sem_i)
            di.wait()
            di2.wait()
            gs = [pltpu.async_copy(tabs[b].at[isrc], msgs[b], sem_g)
                  for b in range(B)]
            for g in gs:
                g.wait()
            ss = [pltpu.async_copy(msgs[b], accs[b].at[idst], sem_s, add=True)
                  for b in range(B)]
            for s_ in ss:
                s_.wait()
            return carry

        lax.fori_loop(0, nch, chunk, 0)
        plsc.subcore_barrier()
        for b in range(B):
            pltpu.sync_copy(accs[b].at[rsl], bounce)
            orow = pl.multiple_of((c * B + b) * Np + row, 8)
            pltpu.sync_copy(bounce, out_hbm.at[pl.ds(orow, SLN)])

    return edge


def _dense_pre(degp, xp):
    def body(degp_ref, xp_ref, y_ref, dinv_ref):
        d = degp_ref[0:1, :] + degp_ref[1:2, :] + 1.0
        dinv = lax.rsqrt(d)
        dinv_ref[...] = dinv
        y_ref[...] = xp_ref[...] * dinv

    return pl.pallas_call(
        body,
        out_shape=(jax.ShapeDtypeStruct(xp.shape, jnp.float32),
                   jax.ShapeDtypeStruct((1, xp.shape[1]), jnp.float32)),
    )(degp, xp)


def _dense_mid(aggp, dinv, xp, w1, b1, w2):
    H = w1.shape[0]

    def body(aggp_ref, dinv_ref, xp_ref, w1_ref, b1_ref, w2_ref, t_ref):
        dv = dinv_ref[...]
        a = (aggp_ref[0] + aggp_ref[1] + xp_ref[...] * dv) * dv

        def jb(j, acc):
            return acc + jnp.maximum(a * w1_ref[j] + b1_ref[j], 0.0) * w2_ref[j]

        sv = lax.fori_loop(0, H, jb, jnp.zeros_like(a))
        t_ref[...] = sv * dv

    smem = pl.BlockSpec(memory_space=pltpu.SMEM)
    vmem = pl.BlockSpec(memory_space=pltpu.VMEM)
    return pl.pallas_call(
        body,
        in_specs=[vmem, vmem, vmem, smem, smem, smem],
        out_shape=jax.ShapeDtypeStruct(xp.shape, jnp.float32),
    )(aggp, dinv, xp, w1, b1, w2)


def _dense_fini(outp, dinv, t, b2):
    def body(outp_ref, dinv_ref, t_ref, b2_ref, o_ref):
        dv = dinv_ref[...]
        o_ref[...] = (outp_ref[0] + outp_ref[1] + t_ref[...]) * dv + b2_ref[0]

    smem = pl.BlockSpec(memory_space=pltpu.SMEM)
    vmem = pl.BlockSpec(memory_space=pltpu.VMEM)
    return pl.pallas_call(
        body,
        in_specs=[vmem, vmem, vmem, smem],
        out_shape=jax.ShapeDtypeStruct(t.shape, jnp.float32),
    )(outp, dinv, t, b2)


def kernel(x, edge_index, W1, b1, W2, b2):
    B, N = x.shape
    E = edge_index.shape[1]
    Np = (N // 256 + 1) * 256
    CH = 5000

    Ep = _pad_up(E, NW * CH)
    if Ep != E:
        pad = jnp.full((2, Ep - E), N, dtype=edge_index.dtype)
        e = jnp.concatenate([edge_index, pad], axis=1)
    else:
        e = edge_index
    src = e[0].reshape(-1)
    dst = e[1].reshape(-1)

    xp = jnp.pad(x.astype(jnp.float32), ((0, 0), (0, Np - N)))
    zeros = jnp.zeros((Np // NS,), jnp.float32)
    ones = jnp.ones((CH,), jnp.float32)

    degp = _deg_kernel(Np, Ep, CH)(dst, ones, zeros).reshape(NC, Np)
    y, dinv = _dense_pre(degp, xp)
    aggp = _edge_kernel(B, Np, Ep, CH)(
        y.reshape(-1), src, dst, zeros).reshape(NC, B, Np)
    t = _dense_mid(aggp, dinv, xp, W1.reshape(-1), b1.reshape(-1),
                   W2.reshape(-1))
    outp = _edge_kernel(B, Np, Ep, CH)(
        t.reshape(-1), src, dst, zeros).reshape(NC, B, Np)
    o = _dense_fini(outp, dinv, t, b2.reshape(-1))
    return o[:, :N]
